# baseline (speedup 1.0000x reference)
import functools

import jax
import jax.numpy as jnp
from jax import lax
from jax.experimental import pallas as pl
from jax.experimental.pallas import tpu as pltpu
from jax.experimental.pallas import tpu_sc as plsc

B = 4096
P = 196
S = 256
V = 256
L = 16
PP = 208

GW = 4 * P
NCH = GW // L
G = 4
BLK = G * GW
NBLK = (S * P) // BLK

NC, NS = 2, 16
NW = NC * NS
BPW = B // NW

_MESH = plsc.VectorSubcoreMesh(
    core_axis_name="c", subcore_axis_name="s", num_cores=NC, num_subcores=NS
)


@functools.partial(
    pl.kernel,
    out_type=jax.ShapeDtypeStruct((B * S * P,), jnp.float32),
    mesh=_MESH,
    scratch_types=[
        pltpu.VMEM((V * S,), jnp.float32),
        pltpu.VMEM((S * P,), jnp.float32),
        pltpu.VMEM((PP,), jnp.int32),
        pltpu.VMEM((GW,), jnp.int32),
        pltpu.VMEM((GW,), jnp.int32),
        pltpu.VMEM((GW,), jnp.int32),
        pltpu.VMEM((BLK,), jnp.float32),
        pltpu.VMEM((BLK,), jnp.float32),
        pltpu.SemaphoreType.DMA,
        pltpu.SemaphoreType.DMA,
    ],
    compiler_params=pltpu.CompilerParams(
        use_tc_tiling_on_sc=False, needs_layout_passes=False
    ),
)
def _sc_embed(x_hbm, tab_hbm, bias_hbm, out_hbm, tab_v, bias_v, xrow_v,
              xg_v, pidx_v, sloc_v, st0, st1, sem0, sem1):
    wid = lax.axis_index("s") * NC + lax.axis_index("c")
    pltpu.sync_copy(tab_hbm, tab_v)
    pltpu.sync_copy(bias_hbm, bias_v)

    stages = (st0, st1)
    sems = (sem0, sem1)
    lanes = lax.iota(jnp.int32, L)

    for c in range(NCH):
        w = lanes + (L * c)
        pidx_v[pl.ds(L * c, L)] = lax.rem(w, P)
        sloc_v[pl.ds(L * c, L)] = lax.div(w, P)

    def wait_stage(h):
        pltpu.make_async_copy(stages[h], out_hbm.at[pl.ds(0, BLK)],
                              sems[h]).wait()

    def b_body(bi, carry):
        b = wid * BPW + bi
        pltpu.sync_copy(x_hbm.at[b], xrow_v)
        for c in range(NCH):
            pv = pidx_v[pl.ds(L * c, L)]
            g = plsc.load_gather(xrow_v, [pv])
            xg_v[pl.ds(L * c, L)] = g * 256 + sloc_v[pl.ds(L * c, L)]

        def blk_body(t, carry2):
            for h in range(2):
                blk = 2 * t + h

                @pl.when((bi > 0) | (t > 0))
                def _():
                    wait_stage(h)

                sadd = blk * (4 * G)
                bbase = pl.multiple_of(blk * BLK, BLK)
                for gg in range(G):
                    for c in range(NCH):
                        o = gg * GW + L * c
                        idx = xg_v[pl.ds(o % GW, L)] + (sadd + 4 * gg)
                        tv = plsc.load_gather(tab_v, [idx])
                        bias = bias_v[pl.ds(bbase + o, L)]
                        stages[h][pl.ds(o, L)] = tv + bias
                obase = pl.multiple_of(b * (S * P) + blk * BLK, BLK)
                pltpu.async_copy(stages[h], out_hbm.at[pl.ds(obase, BLK)],
                                 sems[h])
            return carry2

        lax.fori_loop(0, NBLK // 2, blk_body, 0)
        return carry

    lax.fori_loop(0, BPW, b_body, 0)
    wait_stage(0)
    wait_stage(1)


def kernel(x, x_embed_weight, pos_embed):
    xpad = jnp.pad(x, ((0, 0), (0, PP - P)))
    tab = x_embed_weight.reshape(V * S)
    bias = pos_embed.T.reshape(S * P)
    out = _sc_embed(xpad, tab, bias)
    return out.reshape(B, S, P)


# speedup vs baseline: 1.3127x; 1.3127x over previous
import functools

import jax
import jax.numpy as jnp
from jax import lax
from jax.experimental import pallas as pl
from jax.experimental.pallas import tpu as pltpu
from jax.experimental.pallas import tpu_sc as plsc

B = 4096
P = 196
S = 256
V = 256
L = 16
PP = 208

GW = 4 * P
NCH = GW // L
G = 4
BLK = G * GW
NBLK = (S * P) // BLK

NC, NS = 2, 16
NW = NC * NS
BPW = B // NW

_MESH = plsc.VectorSubcoreMesh(
    core_axis_name="c", subcore_axis_name="s", num_cores=NC, num_subcores=NS
)


@functools.partial(
    pl.kernel,
    out_type=jax.ShapeDtypeStruct((B * S * P,), jnp.float32),
    mesh=_MESH,
    scratch_types=[
        pltpu.VMEM((V * S,), jnp.float32),
        pltpu.VMEM((S * P,), jnp.float32),
        pltpu.VMEM((PP,), jnp.int32),
        pltpu.VMEM((GW,), jnp.int32),
        pltpu.VMEM((GW,), jnp.int32),
        pltpu.VMEM((GW,), jnp.int32),
        pltpu.VMEM((BLK,), jnp.float32),
        pltpu.VMEM((BLK,), jnp.float32),
        pltpu.SemaphoreType.DMA,
        pltpu.SemaphoreType.DMA,
    ],
    compiler_params=pltpu.CompilerParams(
        use_tc_tiling_on_sc=False, needs_layout_passes=False
    ),
)
def _sc_embed(x_hbm, tab_hbm, bias_hbm, out_hbm, tab_v, bias_v, xrow_v,
              xg_v, pidx_v, sloc_v, st0, st1, sem0, sem1):
    wid = lax.axis_index("s") * NC + lax.axis_index("c")
    pltpu.sync_copy(tab_hbm, tab_v)
    pltpu.sync_copy(bias_hbm, bias_v)

    stages = (st0, st1)
    sems = (sem0, sem1)
    lanes = lax.iota(jnp.int32, L)

    for c in range(NCH):
        w = lanes + (L * c)
        pidx_v[pl.ds(L * c, L)] = lax.rem(w, P)
        sloc_v[pl.ds(L * c, L)] = lax.div(w, P)

    def wait_stage(h):
        pltpu.make_async_copy(stages[h], out_hbm.at[pl.ds(0, BLK)],
                              sems[h]).wait()

    def b_body(bi, carry):
        b = wid * BPW + bi
        pltpu.sync_copy(x_hbm.at[b], xrow_v)

        @plsc.parallel_loop(0, NCH, unroll=7)
        def _(c):
            o = pl.multiple_of(c * L, L)
            pv = pidx_v[pl.ds(o, L)]
            g = plsc.load_gather(xrow_v, [pv])
            xg_v[pl.ds(o, L)] = g * 256 + sloc_v[pl.ds(o, L)]

        def blk_body(t, carry2):
            for h in range(2):
                blk = 2 * t + h

                @pl.when((bi > 0) | (t > 0))
                def _():
                    wait_stage(h)

                bbase = pl.multiple_of(blk * BLK, BLK)
                for gg in range(G):
                    sadd = blk * (4 * G) + 4 * gg

                    @plsc.parallel_loop(0, NCH, unroll=7)
                    def _(c):
                        o = pl.multiple_of(c * L, L)
                        idx = xg_v[pl.ds(o, L)] + sadd
                        tv = plsc.load_gather(tab_v, [idx])
                        bias = bias_v[pl.ds(bbase + gg * GW + o, L)]
                        stages[h][pl.ds(gg * GW + o, L)] = tv + bias
                obase = pl.multiple_of(b * (S * P) + blk * BLK, BLK)
                pltpu.async_copy(stages[h], out_hbm.at[pl.ds(obase, BLK)],
                                 sems[h])
            return carry2

        lax.fori_loop(0, NBLK // 2, blk_body, 0)
        return carry

    lax.fori_loop(0, BPW, b_body, 0)
    wait_stage(0)
    wait_stage(1)


def kernel(x, x_embed_weight, pos_embed):
    xpad = jnp.pad(x, ((0, 0), (0, PP - P)))
    tab = x_embed_weight.reshape(V * S)
    bias = pos_embed.T.reshape(S * P)
    out = _sc_embed(xpad, tab, bias)
    return out.reshape(B, S, P)


# speedup vs baseline: 2.1491x; 1.6372x over previous
import functools

import jax
import jax.numpy as jnp
from jax import lax
from jax.experimental import pallas as pl
from jax.experimental.pallas import tpu as pltpu
from jax.experimental.pallas import tpu_sc as plsc

B = 4096
P = 196
S = 256
V = 256
L = 16
PP = 208
NCH = P // L

NC, NS = 2, 16
NW = NC * NS
BPW = B // NW

SB = 16
NSB = S // SB

_MESH = plsc.VectorSubcoreMesh(
    core_axis_name="c", subcore_axis_name="s", num_cores=NC, num_subcores=NS
)


@functools.partial(
    pl.kernel,
    out_type=jax.ShapeDtypeStruct((B, S, P), jnp.float32),
    mesh=_MESH,
    scratch_types=[
        pltpu.VMEM((V * S,), jnp.float32),
        pltpu.VMEM((S * PP,), jnp.float32),
        pltpu.VMEM((PP,), jnp.int32),
        pltpu.VMEM((SB, P), jnp.float32),
        pltpu.VMEM((SB, P), jnp.float32),
        pltpu.SemaphoreType.DMA,
        pltpu.SemaphoreType.DMA,
    ],
    compiler_params=pltpu.CompilerParams(
        use_tc_tiling_on_sc=True, needs_layout_passes=False
    ),
)
def _sc_embed(x_hbm, tab_hbm, bias_hbm, out_hbm, tab_v, bias_v, xrow_v,
              st0, st1, sem0, sem1):
    wid = lax.axis_index("s") * NC + lax.axis_index("c")
    pltpu.sync_copy(tab_hbm, tab_v)
    pltpu.sync_copy(bias_hbm, bias_v)

    stages = (st0, st1)
    sems = (sem0, sem1)
    lanes = lax.iota(jnp.int32, L)
    rem_mask = lanes < (P - L * NCH)
    rem_cols = lanes + (L * NCH)

    def wait_stage(h):
        pltpu.make_async_copy(stages[h], out_hbm.at[0, pl.ds(0, SB), :],
                              sems[h]).wait()

    def b_body(bi, carry):
        b = wid * BPW + bi
        pltpu.sync_copy(x_hbm.at[pl.ds(b * PP, PP)], xrow_v)
        xv = []
        for c in range(NCH + 1):
            g = plsc.load_gather(xrow_v, [lanes + (L * c)])
            xv.append(g * 256)

        def blk_body(t, carry2):
            for h in range(2):
                blk = 2 * t + h
                sbase = blk * SB

                @pl.when((bi > 0) | (t > 0))
                def _():
                    wait_stage(h)

                @plsc.parallel_loop(0, SB, unroll=4)
                def _(j):
                    s = sbase + j
                    boff = pl.multiple_of(s * PP, L)
                    for c in range(NCH):
                        tv = plsc.load_gather(tab_v, [xv[c] + s])
                        bias = bias_v[pl.ds(boff + L * c, L)]
                        stages[h][j, pl.ds(L * c, L)] = tv + bias
                    tv = plsc.load_gather(tab_v, [xv[NCH] + s],
                                          mask=rem_mask)
                    bias = plsc.load_gather(bias_v, [boff + rem_cols],
                                            mask=rem_mask)
                    plsc.store_scatter(stages[h],
                                       [jnp.full((L,), j, jnp.int32),
                                        rem_cols],
                                       tv + bias, mask=rem_mask)

                pltpu.async_copy(stages[h],
                                 out_hbm.at[b, pl.ds(sbase, SB), :],
                                 sems[h])
            return carry2

        lax.fori_loop(0, NSB // 2, blk_body, 0)
        return carry

    lax.fori_loop(0, BPW, b_body, 0)
    wait_stage(0)
    wait_stage(1)


def kernel(x, x_embed_weight, pos_embed):
    xpad = jnp.pad(x, ((0, 0), (0, PP - P))).reshape(B * PP)
    tab = x_embed_weight.reshape(V * S)
    bias = jnp.pad(pos_embed.T, ((0, 0), (0, PP - P))).reshape(S * PP)
    return _sc_embed(xpad, tab, bias)


# speedup vs baseline: 4.9019x; 2.2809x over previous
import functools

import jax
import jax.numpy as jnp
from jax import lax
from jax.experimental import pallas as pl
from jax.experimental.pallas import tpu as pltpu
from jax.experimental.pallas import tpu_sc as plsc

B = 4096
P = 196
S = 256
V = 256
L = 16
PP = 208
NCH = P // L
SP1 = S + 1

NC, NS = 2, 16
NW = NC * NS
BPW = B // NW

SB = 16
NSB = S // SB

_MESH = plsc.VectorSubcoreMesh(
    core_axis_name="c", subcore_axis_name="s", num_cores=NC, num_subcores=NS
)


@functools.partial(
    pl.kernel,
    out_type=jax.ShapeDtypeStruct((B, S, P), jnp.float32),
    mesh=_MESH,
    scratch_types=[
        pltpu.VMEM((V * SP1,), jnp.float32),
        pltpu.VMEM((S * PP,), jnp.float32),
        pltpu.VMEM((PP,), jnp.int32),
        pltpu.VMEM((SB, P), jnp.float32),
        pltpu.VMEM((SB, P), jnp.float32),
        pltpu.SemaphoreType.DMA,
        pltpu.SemaphoreType.DMA,
    ],
    compiler_params=pltpu.CompilerParams(
        use_tc_tiling_on_sc=True, needs_layout_passes=False
    ),
)
def _sc_embed(x_hbm, tab_hbm, bias_hbm, out_hbm, tab_v, bias_v, xrow_v,
              st0, st1, sem0, sem1):
    wid = lax.axis_index("s") * NC + lax.axis_index("c")
    pltpu.sync_copy(tab_hbm, tab_v)
    pltpu.sync_copy(bias_hbm, bias_v)

    stages = (st0, st1)
    sems = (sem0, sem1)
    lanes = lax.iota(jnp.int32, L)
    rem_mask = lanes < (P - L * NCH)
    rem_cols = lanes + (L * NCH)

    def wait_stage(h):
        pltpu.make_async_copy(stages[h], out_hbm.at[0, pl.ds(0, SB), :],
                              sems[h]).wait()

    def b_body(bi, carry):
        b = wid * BPW + bi
        pltpu.sync_copy(x_hbm.at[pl.ds(b * PP, PP)], xrow_v)
        xv = []
        for c in range(NCH + 1):
            xv.append(plsc.load_gather(xrow_v, [lanes + (L * c)]))

        def blk_body(t, carry2):
            for h in range(2):
                blk = 2 * t + h
                sbase = blk * SB

                @pl.when((bi > 0) | (t > 0))
                def _():
                    wait_stage(h)

                @plsc.parallel_loop(0, SB, unroll=4)
                def _(j):
                    s = sbase + j
                    boff = pl.multiple_of(s * PP, L)
                    for c in range(NCH):
                        tv = plsc.load_gather(tab_v, [xv[c] + s])
                        bias = bias_v[pl.ds(boff + L * c, L)]
                        stages[h][j, pl.ds(L * c, L)] = tv + bias
                    tv = plsc.load_gather(tab_v, [xv[NCH] + s],
                                          mask=rem_mask)
                    bias = plsc.load_gather(bias_v, [boff + rem_cols],
                                            mask=rem_mask)
                    plsc.store_scatter(stages[h],
                                       [jnp.full((L,), j, jnp.int32),
                                        rem_cols],
                                       tv + bias, mask=rem_mask)

                pltpu.async_copy(stages[h],
                                 out_hbm.at[b, pl.ds(sbase, SB), :],
                                 sems[h])
            return carry2

        lax.fori_loop(0, NSB // 2, blk_body, 0)
        return carry

    lax.fori_loop(0, BPW, b_body, 0)
    wait_stage(0)
    wait_stage(1)


def kernel(x, x_embed_weight, pos_embed):
    xpad = jnp.pad(x * SP1, ((0, 0), (0, PP - P))).reshape(B * PP)
    tab = jnp.pad(x_embed_weight, ((0, 0), (0, 1))).reshape(V * SP1)
    bias = jnp.pad(pos_embed.T, ((0, 0), (0, PP - P))).reshape(S * PP)
    return _sc_embed(xpad, tab, bias)


# speedup vs baseline: 4.9217x; 1.0040x over previous
import functools

import jax
import jax.numpy as jnp
from jax import lax
from jax.experimental import pallas as pl
from jax.experimental.pallas import tpu as pltpu
from jax.experimental.pallas import tpu_sc as plsc

B = 4096
P = 196
S = 256
V = 256
L = 16
PP = 208
NCH = P // L
SP1 = S + 1

NC, NS = 2, 16
NW = NC * NS
BPW = B // NW

SB = 8
NSB = S // SB

_MESH = plsc.VectorSubcoreMesh(
    core_axis_name="c", subcore_axis_name="s", num_cores=NC, num_subcores=NS
)


@functools.partial(
    pl.kernel,
    out_type=jax.ShapeDtypeStruct((B, S, P), jnp.float32),
    mesh=_MESH,
    scratch_types=[
        pltpu.VMEM((V * SP1,), jnp.float32),
        pltpu.VMEM((S * PP,), jnp.float32),
        pltpu.VMEM((2 * PP,), jnp.int32),
        pltpu.VMEM((2, SB, P), jnp.float32),
        pltpu.VMEM((2, SB, P), jnp.float32),
        pltpu.SemaphoreType.DMA,
        pltpu.SemaphoreType.DMA,
    ],
    compiler_params=pltpu.CompilerParams(
        use_tc_tiling_on_sc=True, needs_layout_passes=False
    ),
)
def _sc_embed(x_hbm, tab_hbm, bias_hbm, out_hbm, tab_v, bias_v, xrow_v,
              st0, st1, sem0, sem1):
    wid = lax.axis_index("s") * NC + lax.axis_index("c")
    pltpu.sync_copy(tab_hbm, tab_v)
    pltpu.sync_copy(bias_hbm, bias_v)

    stages = (st0, st1)
    sems = (sem0, sem1)
    lanes = lax.iota(jnp.int32, L)
    rem_mask = lanes < (P - L * NCH)
    rem_cols = lanes + (L * NCH)

    def wait_stage(h):
        for _ in range(2):
            pltpu.make_async_copy(stages[h].at[0],
                                  out_hbm.at[0, pl.ds(0, SB), :],
                                  sems[h]).wait()

    def b_body(bi, carry):
        b0 = wid * BPW + 2 * bi
        pltpu.sync_copy(x_hbm.at[pl.ds(b0 * PP, 2 * PP)], xrow_v)
        xv = [[plsc.load_gather(xrow_v, [lanes + (L * c + bb * PP)])
               for c in range(NCH + 1)] for bb in range(2)]

        def blk_body(t, carry2):
            for h in range(2):
                blk = 2 * t + h
                sbase = blk * SB

                @pl.when((bi > 0) | (t > 0))
                def _():
                    wait_stage(h)

                @plsc.parallel_loop(0, SB, unroll=4)
                def _(j):
                    s = sbase + j
                    boff = pl.multiple_of(s * PP, L)
                    for c in range(NCH):
                        bias = bias_v[pl.ds(boff + L * c, L)]
                        for bb in range(2):
                            tv = plsc.load_gather(tab_v, [xv[bb][c] + s])
                            stages[h][bb, j, pl.ds(L * c, L)] = tv + bias
                    biasr = plsc.load_gather(bias_v, [boff + rem_cols],
                                             mask=rem_mask)
                    jf = jnp.full((L,), j, jnp.int32)
                    for bb in range(2):
                        tvr = plsc.load_gather(tab_v, [xv[bb][NCH] + s],
                                               mask=rem_mask)
                        plsc.store_scatter(
                            stages[h],
                            [jnp.full((L,), bb, jnp.int32), jf, rem_cols],
                            tvr + biasr, mask=rem_mask)

                for bb in range(2):
                    pltpu.async_copy(stages[h].at[bb],
                                     out_hbm.at[b0 + bb, pl.ds(sbase, SB), :],
                                     sems[h])
            return carry2

        lax.fori_loop(0, NSB // 2, blk_body, 0)
        return carry

    lax.fori_loop(0, BPW // 2, b_body, 0)
    wait_stage(0)
    wait_stage(1)


def kernel(x, x_embed_weight, pos_embed):
    xpad = jnp.pad(x * SP1, ((0, 0), (0, PP - P))).reshape(B * PP)
    tab = jnp.pad(x_embed_weight, ((0, 0), (0, 1))).reshape(V * SP1)
    bias = jnp.pad(pos_embed.T, ((0, 0), (0, PP - P))).reshape(S * PP)
    return _sc_embed(xpad, tab, bias)
